# Initial kernel scaffold; baseline (speedup 1.0000x reference)
import jax
import jax.numpy as jnp
from jax.experimental import pallas as pl


def kernel(patch, pos_table):
    raise NotImplementedError("write your pallas kernel here")


# speedup vs baseline: 1.0150x; 1.0150x over previous
import jax
import jax.numpy as jnp
from jax.experimental import pallas as pl


def _add_body(patch_ref, pos_ref, out_ref):
    out_ref[...] = patch_ref[...] + pos_ref[...]


def kernel(patch, pos_table):
    batch, num_patches, proj_dim = patch.shape
    return pl.pallas_call(
        _add_body,
        grid=(batch,),
        in_specs=[
            pl.BlockSpec((1, num_patches, proj_dim), lambda b: (b, 0, 0)),
            pl.BlockSpec((num_patches, proj_dim), lambda b: (0, 0)),
        ],
        out_specs=pl.BlockSpec((1, num_patches, proj_dim), lambda b: (b, 0, 0)),
        out_shape=jax.ShapeDtypeStruct(patch.shape, patch.dtype),
    )(patch, pos_table)
